# baseline (speedup 1.0000x reference)
import jax
import jax.numpy as jnp
from jax.experimental import pallas as pl
from jax.experimental.pallas import tpu as pltpu


def _fused_body(*refs):
    (aref, w1ref, b1ref, w2ref, b2ref, *word_refs,
     logits_ref, x_ref, sims_ref, an_scratch, ones_scratch) = refs
    i = pl.program_id(0)

    @pl.when(i == 0)
    def _():
        a = aref[...]
        norm = jnp.sqrt(jnp.sum(a * a, axis=1, keepdims=True))
        an_scratch[...] = a / (norm + 1e-8)
        ones_scratch[...] = jnp.ones_like(ones_scratch)

    w = len(word_refs)
    vals = [r[:, 0, 0, :] for r in word_refs]
    while len(vals) > 1:
        vals = [vals[j] + vals[j + 1] for j in range(0, len(vals), 2)]
    x = vals[0] * (1.0 / w)
    x_ref[...] = x

    nsq = jax.lax.dot_general(
        x * x, ones_scratch[...],
        dimension_numbers=(((1,), (0,)), ((), ())),
        preferred_element_type=jnp.float32)[:, 0:1]
    inv = 1.0 / (jnp.sqrt(nsq) + 1e-8)
    s0 = jax.lax.dot_general(
        x, an_scratch[...],
        dimension_numbers=(((1,), (1,)), ((), ())),
        preferred_element_type=jnp.float32)
    sims = s0 * inv
    sims_ref[...] = sims

    h = sims @ w1ref[...] + b1ref[...]
    h = h * jax.nn.sigmoid(h)
    out = h @ w2ref[...] + b2ref[...]
    logits_ref[...] = jnp.mean(out, axis=0, keepdims=True)[None]


def kernel(encodings, words_per_sentence, sentences_per_text,
           anchor_samples, W1, b1, W2, b2):
    total_tokens, d = encodings.shape
    n_sent = words_per_sentence.shape[0]
    n_text = sentences_per_text.shape[0]
    words = total_tokens // n_sent
    sent_per_text = n_sent // n_text
    n_anchors = anchor_samples.shape[0]
    hid = W1.shape[1]
    n_classes = W2.shape[1]

    pad_c = 128 - n_classes
    W2p = jnp.pad(W2, ((0, 0), (0, pad_c)))
    b2p = jnp.pad(b2, ((0, pad_c),)).reshape(1, 128)
    b1r = b1.reshape(1, hid)
    e4 = encodings.reshape(n_sent, words, 1, d)

    word_specs = [
        pl.BlockSpec((sent_per_text, 1, 1, d), lambda i, w=w: (i, w, 0, 0))
        for w in range(words)
    ]

    grid = (n_text,)
    logits_pad, x, sims = pl.pallas_call(
        _fused_body,
        grid=grid,
        in_specs=[
            pl.BlockSpec((n_anchors, d), lambda i: (0, 0)),
            pl.BlockSpec((d, hid), lambda i: (0, 0)),
            pl.BlockSpec((1, hid), lambda i: (0, 0)),
            pl.BlockSpec((hid, 128), lambda i: (0, 0)),
            pl.BlockSpec((1, 128), lambda i: (0, 0)),
        ] + word_specs,
        out_specs=[
            pl.BlockSpec((1, 1, 128), lambda i: (i, 0, 0)),
            pl.BlockSpec((sent_per_text, d), lambda i: (i, 0)),
            pl.BlockSpec((sent_per_text, n_anchors), lambda i: (i, 0)),
        ],
        out_shape=[
            jax.ShapeDtypeStruct((n_text, 1, 128), jnp.float32),
            jax.ShapeDtypeStruct((n_sent, d), jnp.float32),
            jax.ShapeDtypeStruct((n_sent, n_anchors), jnp.float32),
        ],
        scratch_shapes=[pltpu.VMEM((n_anchors, d), jnp.float32),
                        pltpu.VMEM((d, 128), jnp.float32)],
    )(anchor_samples, W1, b1r, W2p, b2p, *([e4] * words))

    logits = logits_pad.reshape(n_text, 128)[:, :n_classes]
    return (logits, x, sims)


# speedup vs baseline: 7.7292x; 7.7292x over previous
import jax
import jax.numpy as jnp
from jax.experimental import pallas as pl
from jax.experimental.pallas import tpu as pltpu


def _fused_body(eref, aref, w1ref, b1ref, w2ref, b2ref,
                logits_ref, x_ref, sims_ref,
                an_scratch, ones_scratch, msum_scratch):
    i = pl.program_id(0)

    @pl.when(i == 0)
    def _():
        a = aref[...]
        norm = jnp.sqrt(jnp.sum(a * a, axis=1, keepdims=True))
        an_scratch[...] = a / (norm + 1e-8)
        ones_scratch[...] = jnp.ones_like(ones_scratch)
        sblk, cols = msum_scratch.shape
        rows_id = jax.lax.broadcasted_iota(jnp.int32, (sblk, cols), 0)
        cols_id = jax.lax.broadcasted_iota(jnp.int32, (sblk, cols), 1)
        w = 2 * cols // sblk
        msum_scratch[...] = jnp.where(
            cols_id // (cols // sblk) == rows_id, 1.0 / w, 0.0)

    e = eref[...]
    sblk = msum_scratch.shape[0]
    w = e.shape[0] // sblk
    d = e.shape[1]
    er = e.reshape(sblk, 2, w // 2, d)
    g = (er[:, 0, :, :] + er[:, 1, :, :]).reshape(sblk * (w // 2), d)
    x = jax.lax.dot_general(
        msum_scratch[...], g,
        dimension_numbers=(((1,), (0,)), ((), ())),
        preferred_element_type=jnp.float32)
    x_ref[...] = x

    nsq = jax.lax.dot_general(
        x * x, ones_scratch[...],
        dimension_numbers=(((1,), (0,)), ((), ())),
        preferred_element_type=jnp.float32)[:, 0:1]
    inv = 1.0 / (jnp.sqrt(nsq) + 1e-8)
    s0 = jax.lax.dot_general(
        x, an_scratch[...],
        dimension_numbers=(((1,), (1,)), ((), ())),
        preferred_element_type=jnp.float32)
    sims = s0 * inv
    sims_ref[...] = sims

    h = sims @ w1ref[...] + b1ref[...]
    h = h * jax.nn.sigmoid(h)
    out = h @ w2ref[...] + b2ref[...]
    logits_ref[...] = jnp.mean(out, axis=0, keepdims=True)[None]


def kernel(encodings, words_per_sentence, sentences_per_text,
           anchor_samples, W1, b1, W2, b2):
    total_tokens, d = encodings.shape
    n_sent = words_per_sentence.shape[0]
    n_text = sentences_per_text.shape[0]
    words = total_tokens // n_sent
    sent_per_text = n_sent // n_text
    n_anchors = anchor_samples.shape[0]
    hid = W1.shape[1]
    n_classes = W2.shape[1]

    pad_c = 128 - n_classes
    W2p = jnp.pad(W2, ((0, 0), (0, pad_c)))
    b2p = jnp.pad(b2, ((0, pad_c),)).reshape(1, 128)
    b1r = b1.reshape(1, hid)

    tok_blk = sent_per_text * words

    grid = (n_text,)
    logits_pad, x, sims = pl.pallas_call(
        _fused_body,
        grid=grid,
        in_specs=[
            pl.BlockSpec((tok_blk, d), lambda i: (i, 0)),
            pl.BlockSpec((n_anchors, d), lambda i: (0, 0)),
            pl.BlockSpec((d, hid), lambda i: (0, 0)),
            pl.BlockSpec((1, hid), lambda i: (0, 0)),
            pl.BlockSpec((hid, 128), lambda i: (0, 0)),
            pl.BlockSpec((1, 128), lambda i: (0, 0)),
        ],
        out_specs=[
            pl.BlockSpec((1, 1, 128), lambda i: (i, 0, 0)),
            pl.BlockSpec((sent_per_text, d), lambda i: (i, 0)),
            pl.BlockSpec((sent_per_text, n_anchors), lambda i: (i, 0)),
        ],
        out_shape=[
            jax.ShapeDtypeStruct((n_text, 1, 128), jnp.float32),
            jax.ShapeDtypeStruct((n_sent, d), jnp.float32),
            jax.ShapeDtypeStruct((n_sent, n_anchors), jnp.float32),
        ],
        scratch_shapes=[
            pltpu.VMEM((n_anchors, d), jnp.float32),
            pltpu.VMEM((d, 128), jnp.float32),
            pltpu.VMEM((sent_per_text, sent_per_text * words // 2),
                       jnp.float32),
        ],
    )(encodings, anchor_samples, W1, b1r, W2p, b2p)

    logits = logits_pad.reshape(n_text, 128)[:, :n_classes]
    return (logits, x, sims)


# speedup vs baseline: 8.3101x; 1.0752x over previous
import jax
import jax.numpy as jnp
from jax.experimental import pallas as pl
from jax.experimental.pallas import tpu as pltpu


def _fused_body(eref, aref, w1ref, b1ref, w2ref, b2ref,
                logits_ref, x_ref, sims_ref,
                an_scratch, ones_scratch, msum_scratch):
    i = pl.program_id(0)

    @pl.when(i == 0)
    def _():
        a = aref[...]
        norm = jnp.sqrt(jnp.sum(a * a, axis=1, keepdims=True))
        an_scratch[...] = a / (norm + 1e-8)
        ones_scratch[...] = jnp.ones_like(ones_scratch)
        sblk, cols = msum_scratch.shape
        rows_id = jax.lax.broadcasted_iota(jnp.int32, (sblk, cols), 0)
        cols_id = jax.lax.broadcasted_iota(jnp.int32, (sblk, cols), 1)
        w = 2 * cols // sblk
        msum_scratch[...] = jnp.where(
            cols_id // (cols // sblk) == rows_id, 1.0 / w, 0.0)

    e = eref[...]
    sblk = msum_scratch.shape[0]
    w = e.shape[0] // sblk
    d = e.shape[1]
    er = e.reshape(sblk, 2, w // 2, d)
    g = (er[:, 0, :, :] + er[:, 1, :, :]).reshape(sblk * (w // 2), d)
    x = jax.lax.dot_general(
        msum_scratch[...], g,
        dimension_numbers=(((1,), (0,)), ((), ())),
        preferred_element_type=jnp.float32)
    x_ref[...] = x

    nsq = jax.lax.dot_general(
        x * x, ones_scratch[...],
        dimension_numbers=(((1,), (0,)), ((), ())),
        preferred_element_type=jnp.float32)[:, 0:1]
    inv = 1.0 / (jnp.sqrt(nsq) + 1e-8)
    s0 = jax.lax.dot_general(
        x, an_scratch[...],
        dimension_numbers=(((1,), (1,)), ((), ())),
        preferred_element_type=jnp.float32)
    sims = s0 * inv
    sims_ref[...] = sims

    h = sims @ w1ref[...] + b1ref[...]
    h = h * jax.nn.sigmoid(h)
    out = h @ w2ref[...] + b2ref[...]
    tps = logits_ref.shape[0]
    logits_ref[...] = jnp.mean(
        out.reshape(tps, out.shape[0] // tps, out.shape[1]), axis=1,
        keepdims=True)


def kernel(encodings, words_per_sentence, sentences_per_text,
           anchor_samples, W1, b1, W2, b2):
    total_tokens, d = encodings.shape
    n_sent = words_per_sentence.shape[0]
    n_text = sentences_per_text.shape[0]
    words = total_tokens // n_sent
    sent_per_text = n_sent // n_text
    n_anchors = anchor_samples.shape[0]
    hid = W1.shape[1]
    n_classes = W2.shape[1]

    pad_c = 128 - n_classes
    W2p = jnp.pad(W2, ((0, 0), (0, pad_c)))
    b2p = jnp.pad(b2, ((0, pad_c),)).reshape(1, 128)
    b1r = b1.reshape(1, hid)

    texts_per_step = 2
    s_blk = texts_per_step * sent_per_text
    tok_blk = s_blk * words
    grid = (n_text // texts_per_step,)
    logits_pad, x, sims = pl.pallas_call(
        _fused_body,
        grid=grid,
        in_specs=[
            pl.BlockSpec((tok_blk, d), lambda i: (i, 0)),
            pl.BlockSpec((n_anchors, d), lambda i: (0, 0)),
            pl.BlockSpec((d, hid), lambda i: (0, 0)),
            pl.BlockSpec((1, hid), lambda i: (0, 0)),
            pl.BlockSpec((hid, 128), lambda i: (0, 0)),
            pl.BlockSpec((1, 128), lambda i: (0, 0)),
        ],
        out_specs=[
            pl.BlockSpec((texts_per_step, 1, 128), lambda i: (i, 0, 0)),
            pl.BlockSpec((s_blk, d), lambda i: (i, 0)),
            pl.BlockSpec((s_blk, n_anchors), lambda i: (i, 0)),
        ],
        out_shape=[
            jax.ShapeDtypeStruct((n_text, 1, 128), jnp.float32),
            jax.ShapeDtypeStruct((n_sent, d), jnp.float32),
            jax.ShapeDtypeStruct((n_sent, n_anchors), jnp.float32),
        ],
        scratch_shapes=[
            pltpu.VMEM((n_anchors, d), jnp.float32),
            pltpu.VMEM((d, 128), jnp.float32),
            pltpu.VMEM((s_blk, s_blk * words // 2), jnp.float32),
        ],
    )(encodings, anchor_samples, W1, b1r, W2p, b2p)

    logits = logits_pad.reshape(n_text, 128)[:, :n_classes]
    return (logits, x, sims)
